# baseline (speedup 1.0000x reference)
import functools

import jax
import jax.numpy as jnp
from jax import lax
from jax.experimental import pallas as pl
from jax.experimental.pallas import tpu as pltpu
from jax.experimental.pallas import tpu_sc as plsc
from jax.experimental import layout as jlayout

BATCH = 4096
SEQ = 200
DIM = 128
VOCAB = 100000
CHUNK = 40
NCHUNK = SEQ // CHUNK
NGRP = DIM // 16


def _tanh(x):
    e = jnp.exp(-2.0 * jnp.abs(x))
    t = (1.0 - e) / (1.0 + e)
    return jnp.where(x < 0, -t, t)


@functools.partial(jax.jit, static_argnums=(2, 3))
def _emb_maxpool(idx_flat, table, nc, ns):
    nw = nc * ns
    bpw = BATCH // nw
    ipw = bpw * SEQ

    mesh = plsc.VectorSubcoreMesh(core_axis_name="c", subcore_axis_name="s")

    @functools.partial(
        pl.kernel,
        out_type=jax.ShapeDtypeStruct((BATCH, DIM), jnp.float32),
        mesh=mesh,
        compiler_params=pltpu.CompilerParams(use_tc_tiling_on_sc=False),
        scratch_types=[
            pltpu.VMEM((ipw,), jnp.int32),
            pltpu.VMEM((SEQ, DIM), jnp.float32),
            pltpu.VMEM((SEQ, DIM), jnp.float32),
            pltpu.VMEM((bpw, DIM), jnp.float32),
            pltpu.SemaphoreType.DMA,
            pltpu.SemaphoreType.DMA,
        ],
    )
    def k(idx_hbm, table_hbm, out_hbm, idx_v, buf0, buf1, out_v, sem0, sem1):
        wid = lax.axis_index("s") * nc + lax.axis_index("c")
        base = wid * bpw
        tbl = table_hbm
        pltpu.sync_copy(idx_hbm.at[pl.ds(wid * ipw, ipw)], idx_v)

        bufs = (buf0, buf1)
        sems = (sem0, sem1)

        def start(r, b):
            for j in range(NCHUNK):
                pltpu.async_copy(
                    tbl.at[idx_v.at[pl.ds(r * SEQ + j * CHUNK, CHUNK)]],
                    bufs[b].at[pl.ds(j * CHUNK, CHUNK)],
                    sems[b],
                )

        def wait(r, b):
            for j in range(NCHUNK):
                pltpu.make_async_copy(
                    tbl.at[idx_v.at[pl.ds(r * SEQ + j * CHUNK, CHUNK)]],
                    bufs[b].at[pl.ds(j * CHUNK, CHUNK)],
                    sems[b],
                ).wait()

        start(0, 0)
        start(1, 1)

        @pl.loop(0, bpw, step=2)
        def _rows(g):
            for b in range(2):
                r = g + b
                wait(r, b)
                buf = bufs[b]

                def body(s, accs):
                    return tuple(
                        jnp.maximum(accs[d], buf[s, pl.ds(d * 16, 16)])
                        for d in range(NGRP)
                    )

                ninf = tuple(
                    jnp.full((16,), -jnp.inf, jnp.float32) for _ in range(NGRP)
                )
                acc = lax.fori_loop(0, SEQ, body, ninf, unroll=8)
                for d in range(NGRP):
                    out_v[r, pl.ds(d * 16, 16)] = _tanh(acc[d])

                nxt = r + 2
                @pl.when(nxt < bpw)
                def _():
                    start(nxt, b)

        pltpu.sync_copy(out_v, out_hbm.at[pl.ds(base, bpw)])

    return k(idx_flat, table)


def kernel(input, table):
    info = plsc.get_sparse_core_info()
    idx_flat = input.reshape(BATCH * SEQ)
    idx_flat = jlayout.with_layout_constraint(
        idx_flat, jlayout.Layout((0,), ())
    )
    table = jlayout.with_layout_constraint(table, jlayout.Layout((0, 1), ()))
    return _emb_maxpool(idx_flat, table, info.num_cores, info.num_subcores)


# speedup vs baseline: 1.2350x; 1.2350x over previous
import functools

import jax
import jax.numpy as jnp
from jax import lax
from jax.experimental import pallas as pl
from jax.experimental.pallas import tpu as pltpu
from jax.experimental.pallas import tpu_sc as plsc
from jax.experimental import layout as jlayout

BATCH = 4096
SEQ = 200
DIM = 128
VOCAB = 100000
CHUNK = 40
NCHUNK = SEQ // CHUNK
NGRP = DIM // 16


def _tanh(x):
    e = jnp.exp(-2.0 * jnp.abs(x))
    t = (1.0 - e) / (1.0 + e)
    return jnp.where(x < 0, -t, t)


@functools.partial(jax.jit, static_argnums=(2, 3))
def _emb_maxpool(idx_flat, table, nc, ns):
    nw = nc * ns
    bpw = BATCH // nw
    ipw = bpw * SEQ

    mesh = plsc.VectorSubcoreMesh(core_axis_name="c", subcore_axis_name="s")

    @functools.partial(
        pl.kernel,
        out_type=jax.ShapeDtypeStruct((BATCH, DIM), jnp.float32),
        mesh=mesh,
        compiler_params=pltpu.CompilerParams(use_tc_tiling_on_sc=False),
        scratch_types=[
            pltpu.VMEM((ipw,), jnp.int32),
            pltpu.VMEM((SEQ, DIM), jnp.float32),
            pltpu.VMEM((SEQ, DIM), jnp.float32),
            pltpu.VMEM((SEQ, DIM), jnp.float32),
            pltpu.VMEM((bpw, DIM), jnp.float32),
            pltpu.SemaphoreType.DMA,
            pltpu.SemaphoreType.DMA,
            pltpu.SemaphoreType.DMA,
        ],
    )
    def k(
        idx_hbm, table_hbm, out_hbm,
        idx_v, buf0, buf1, buf2, out_v, sem0, sem1, sem2,
    ):
        wid = lax.axis_index("s") * nc + lax.axis_index("c")
        base = wid * bpw
        tbl = table_hbm
        pltpu.sync_copy(idx_hbm.at[pl.ds(wid * ipw, ipw)], idx_v)

        bufs = (buf0, buf1, buf2)
        sems = (sem0, sem1, sem2)

        def start(r, b):
            for j in range(NCHUNK):
                pltpu.async_copy(
                    tbl.at[idx_v.at[pl.ds(r * SEQ + j * CHUNK, CHUNK)]],
                    bufs[b].at[pl.ds(j * CHUNK, CHUNK)],
                    sems[b],
                )

        def wait(r, b):
            for j in range(NCHUNK):
                pltpu.make_async_copy(
                    tbl.at[idx_v.at[pl.ds(r * SEQ + j * CHUNK, CHUNK)]],
                    bufs[b].at[pl.ds(j * CHUNK, CHUNK)],
                    sems[b],
                ).wait()

        def process(r, b):
            wait(r, b)
            buf = bufs[b]

            def body(s, accs):
                return tuple(
                    jnp.maximum(accs[d], buf[s, pl.ds(d * 16, 16)])
                    for d in range(NGRP)
                )

            ninf = tuple(
                jnp.full((16,), -jnp.inf, jnp.float32) for _ in range(NGRP)
            )
            acc = lax.fori_loop(0, SEQ, body, ninf, unroll=8)
            for d in range(NGRP):
                out_v[r, pl.ds(d * 16, 16)] = _tanh(acc[d])

            nxt = r + 3
            @pl.when(nxt < bpw)
            def _():
                start(nxt, b)

        for b in range(3):
            start(b, b)

        nmain = (bpw // 3) * 3

        @pl.loop(0, nmain, step=3)
        def _rows(g):
            for b in range(3):
                process(g + b, b)

        for r in range(nmain, bpw):
            process(r, r - nmain)

        pltpu.sync_copy(out_v, out_hbm.at[pl.ds(base, bpw)])

    return k(idx_flat, table)


def kernel(input, table):
    info = plsc.get_sparse_core_info()
    idx_flat = input.reshape(BATCH * SEQ)
    idx_flat = jlayout.with_layout_constraint(
        idx_flat, jlayout.Layout((0,), ())
    )
    table = jlayout.with_layout_constraint(table, jlayout.Layout((0, 1), ()))
    return _emb_maxpool(idx_flat, table, info.num_cores, info.num_subcores)


# speedup vs baseline: 1.2388x; 1.0031x over previous
import functools

import jax
import jax.numpy as jnp
from jax import lax
from jax.experimental import pallas as pl
from jax.experimental.pallas import tpu as pltpu
from jax.experimental.pallas import tpu_sc as plsc
from jax.experimental import layout as jlayout

BATCH = 4096
SEQ = 200
DIM = 128
VOCAB = 100000
CHUNK = 40
NCHUNK = SEQ // CHUNK
NGRP = DIM // 16
DEPTH = 4


def _tanh(x):
    e = jnp.exp(-2.0 * jnp.abs(x))
    t = (1.0 - e) / (1.0 + e)
    return jnp.where(x < 0, -t, t)


@functools.partial(jax.jit, static_argnums=(2, 3))
def _emb_maxpool(idx_flat, table, nc, ns):
    nw = nc * ns
    bpw = BATCH // nw
    ipw = bpw * SEQ

    mesh = plsc.VectorSubcoreMesh(core_axis_name="c", subcore_axis_name="s")

    @functools.partial(
        pl.kernel,
        out_type=jax.ShapeDtypeStruct((BATCH, DIM), jnp.float32),
        mesh=mesh,
        compiler_params=pltpu.CompilerParams(use_tc_tiling_on_sc=False),
        scratch_types=[
            pltpu.VMEM((ipw,), jnp.int32),
            pltpu.VMEM((SEQ, DIM), jnp.float32),
            pltpu.VMEM((SEQ, DIM), jnp.float32),
            pltpu.VMEM((SEQ, DIM), jnp.float32),
            pltpu.VMEM((SEQ, DIM), jnp.float32),
            pltpu.VMEM((DEPTH, DIM), jnp.float32),
            pltpu.SemaphoreType.DMA,
            pltpu.SemaphoreType.DMA,
            pltpu.SemaphoreType.DMA,
            pltpu.SemaphoreType.DMA,
            pltpu.SemaphoreType.DMA,
            pltpu.SemaphoreType.DMA,
            pltpu.SemaphoreType.DMA,
            pltpu.SemaphoreType.DMA,
        ],
    )
    def k(
        idx_hbm, table_hbm, out_hbm,
        idx_v, buf0, buf1, buf2, buf3, oring,
        sem0, sem1, sem2, sem3, osem0, osem1, osem2, osem3,
    ):
        wid = lax.axis_index("s") * nc + lax.axis_index("c")
        base = wid * bpw
        tbl = table_hbm
        pltpu.sync_copy(idx_hbm.at[pl.ds(wid * ipw, ipw)], idx_v)

        bufs = (buf0, buf1, buf2, buf3)
        sems = (sem0, sem1, sem2, sem3)
        osems = (osem0, osem1, osem2, osem3)

        def start(r, b):
            for j in range(NCHUNK):
                pltpu.async_copy(
                    tbl.at[idx_v.at[pl.ds(r * SEQ + j * CHUNK, CHUNK)]],
                    bufs[b].at[pl.ds(j * CHUNK, CHUNK)],
                    sems[b],
                )

        def wait(r, b):
            for j in range(NCHUNK):
                pltpu.make_async_copy(
                    tbl.at[idx_v.at[pl.ds(r * SEQ + j * CHUNK, CHUNK)]],
                    bufs[b].at[pl.ds(j * CHUNK, CHUNK)],
                    sems[b],
                ).wait()

        def owrite(r, b):
            pltpu.async_copy(
                oring.at[pl.ds(b, 1)],
                out_hbm.at[pl.ds(base + r, 1)],
                osems[b],
            )

        def owait(r, b):
            pltpu.make_async_copy(
                oring.at[pl.ds(b, 1)],
                out_hbm.at[pl.ds(base + r, 1)],
                osems[b],
            ).wait()

        def process(r, b):
            wait(r, b)
            buf = bufs[b]

            def body(s, accs):
                return tuple(
                    jnp.maximum(accs[d], buf[s, pl.ds(d * 16, 16)])
                    for d in range(NGRP)
                )

            ninf = tuple(
                jnp.full((16,), -jnp.inf, jnp.float32) for _ in range(NGRP)
            )
            acc = lax.fori_loop(0, SEQ, body, ninf, unroll=8)

            @pl.when(r >= DEPTH)
            def _():
                owait(r - DEPTH, b)

            for d in range(NGRP):
                oring[b, pl.ds(d * 16, 16)] = _tanh(acc[d])
            owrite(r, b)

            nxt = r + DEPTH
            @pl.when(nxt < bpw)
            def _():
                start(nxt, b)

        for b in range(DEPTH):
            start(b, b)

        @pl.loop(0, bpw, step=DEPTH)
        def _rows(g):
            for b in range(DEPTH):
                process(g + b, b)

        for b in range(DEPTH):
            owait(bpw - DEPTH + b, b)

    return k(idx_flat, table)


def kernel(input, table):
    info = plsc.get_sparse_core_info()
    idx_flat = input.reshape(BATCH * SEQ)
    idx_flat = jlayout.with_layout_constraint(
        idx_flat, jlayout.Layout((0,), ())
    )
    table = jlayout.with_layout_constraint(table, jlayout.Layout((0, 1), ()))
    return _emb_maxpool(idx_flat, table, info.num_cores, info.num_subcores)


# speedup vs baseline: 1.2419x; 1.0025x over previous
import functools

import jax
import jax.numpy as jnp
from jax import lax
from jax.experimental import pallas as pl
from jax.experimental.pallas import tpu as pltpu
from jax.experimental.pallas import tpu_sc as plsc
from jax.experimental import layout as jlayout

BATCH = 4096
SEQ = 200
DIM = 128
VOCAB = 100000
CHUNK = 40
NCHUNK = SEQ // CHUNK
NGRP = DIM // 16
DEPTH = 4


def _tanh(x):
    e = jnp.exp(-2.0 * jnp.abs(x))
    t = (1.0 - e) / (1.0 + e)
    return jnp.where(x < 0, -t, t)


@functools.partial(jax.jit, static_argnums=(2, 3))
def _emb_maxpool(idx_flat, table, nc, ns):
    nw = nc * ns
    bpw = BATCH // nw
    ipw = bpw * SEQ

    mesh = plsc.VectorSubcoreMesh(core_axis_name="c", subcore_axis_name="s")

    @functools.partial(
        pl.kernel,
        out_type=jax.ShapeDtypeStruct((BATCH, DIM), jnp.float32),
        mesh=mesh,
        compiler_params=pltpu.CompilerParams(use_tc_tiling_on_sc=False),
        scratch_types=[
            pltpu.VMEM((ipw,), jnp.int32),
            pltpu.VMEM((SEQ, DIM), jnp.float32),
            pltpu.VMEM((SEQ, DIM), jnp.float32),
            pltpu.VMEM((SEQ, DIM), jnp.float32),
            pltpu.VMEM((SEQ, DIM), jnp.float32),
            pltpu.VMEM((DEPTH, DIM), jnp.float32),
            pltpu.SemaphoreType.DMA,
            pltpu.SemaphoreType.DMA,
            pltpu.SemaphoreType.DMA,
            pltpu.SemaphoreType.DMA,
            pltpu.SemaphoreType.DMA,
            pltpu.SemaphoreType.DMA,
            pltpu.SemaphoreType.DMA,
            pltpu.SemaphoreType.DMA,
        ],
    )
    def k(
        idx_hbm, table_hbm, out_hbm,
        idx_v, buf0, buf1, buf2, buf3, oring,
        sem0, sem1, sem2, sem3, osem0, osem1, osem2, osem3,
    ):
        wid = lax.axis_index("s") * nc + lax.axis_index("c")
        base = wid * bpw
        tbl = table_hbm
        pltpu.sync_copy(idx_hbm.at[pl.ds(wid * ipw, ipw)], idx_v)

        bufs = (buf0, buf1, buf2, buf3)
        sems = (sem0, sem1, sem2, sem3)
        osems = (osem0, osem1, osem2, osem3)

        def start(r, b):
            for j in range(NCHUNK):
                pltpu.async_copy(
                    tbl.at[idx_v.at[pl.ds(r * SEQ + j * CHUNK, CHUNK)]],
                    bufs[b].at[pl.ds(j * CHUNK, CHUNK)],
                    sems[b],
                )

        def wait(r, b):
            pltpu.make_async_copy(
                tbl.at[pl.ds(0, SEQ)], bufs[b], sems[b]
            ).wait()

        def owrite(r, b):
            pltpu.async_copy(
                oring.at[pl.ds(b, 1)],
                out_hbm.at[pl.ds(base + r, 1)],
                osems[b],
            )

        def owait(r, b):
            pltpu.make_async_copy(
                oring.at[pl.ds(b, 1)],
                out_hbm.at[pl.ds(base + r, 1)],
                osems[b],
            ).wait()

        def process(r, b):
            wait(r, b)
            buf = bufs[b]

            def body(s, accs):
                return tuple(
                    jnp.maximum(accs[d], buf[s, pl.ds(d * 16, 16)])
                    for d in range(NGRP)
                )

            ninf = tuple(
                jnp.full((16,), -jnp.inf, jnp.float32) for _ in range(NGRP)
            )
            acc = lax.fori_loop(0, SEQ, body, ninf, unroll=8)

            @pl.when(r >= DEPTH)
            def _():
                owait(r - DEPTH, b)

            for d in range(NGRP):
                oring[b, pl.ds(d * 16, 16)] = _tanh(acc[d])
            owrite(r, b)

            nxt = r + DEPTH
            @pl.when(nxt < bpw)
            def _():
                start(nxt, b)

        for b in range(DEPTH):
            start(b, b)

        @pl.loop(0, bpw, step=DEPTH)
        def _rows(g):
            for b in range(DEPTH):
                process(g + b, b)

        for b in range(DEPTH):
            owait(bpw - DEPTH + b, b)

    return k(idx_flat, table)


def kernel(input, table):
    info = plsc.get_sparse_core_info()
    idx_flat = input.reshape(BATCH * SEQ)
    idx_flat = jlayout.with_layout_constraint(
        idx_flat, jlayout.Layout((0,), ())
    )
    table = jlayout.with_layout_constraint(table, jlayout.Layout((0, 1), ()))
    return _emb_maxpool(idx_flat, table, info.num_cores, info.num_subcores)
